# baseline (speedup 1.0000x reference)
import functools

import jax
import jax.numpy as jnp
from jax import lax
from jax.experimental import pallas as pl
from jax.experimental.pallas import tpu as pltpu
from jax.experimental.pallas import tpu_sc as plsc

N_NEURONS = 1000
EMBED_DIM = 128
BATCH = 1024
N_SELECTED = 16384

NC, NS, L = 2, 16, 16
NW = NC * NS
B_PER_W = N_SELECTED // NW
NPAD = 1024


def _sc_emb_body(table_hbm, keys_hbm, emb_hbm, idx_v, rows_v, sem):
    wid = lax.axis_index("s") * NC + lax.axis_index("c")
    base = wid * B_PER_W
    pltpu.sync_copy(keys_hbm.at[pl.ds(base, B_PER_W)], idx_v)
    pltpu.async_copy(table_hbm.at[idx_v], rows_v, sem).wait()
    pltpu.sync_copy(rows_v, emb_hbm.at[pl.ds(base, B_PER_W)])


@functools.cache
def _sc_emb():
    return pl.kernel(
        _sc_emb_body,
        out_type=jax.ShapeDtypeStruct((N_SELECTED, EMBED_DIM), jnp.float32),
        mesh=plsc.VectorSubcoreMesh(core_axis_name="c", subcore_axis_name="s",
                                    num_cores=NC, num_subcores=NS),
        scratch_types=[
            pltpu.VMEM((B_PER_W,), jnp.int32),
            pltpu.VMEM((B_PER_W, EMBED_DIM), jnp.float32),
            pltpu.SemaphoreType.DMA,
        ],
    )


ROW_BLK = 512
COL_BLK = 4096
OH_CHUNK = 1024


def _tc_affine_body(x_ref, sb_ref, k_ref, o_ref, coef_ref):
    @pl.when(pl.program_id(1) == 0)
    def _():
        for c in range(COL_BLK // OH_CHUNK):
            kc = k_ref[:, pl.ds(c * OH_CHUNK, OH_CHUNK)]
            rows = lax.broadcasted_iota(jnp.int32, (NPAD, OH_CHUNK), 0)
            oh = jnp.where(rows == kc, 1.0, 0.0)
            coef_ref[:, pl.ds(c * OH_CHUNK, OH_CHUNK)] = lax.dot_general(
                sb_ref[...], oh, (((1,), (0,)), ((), ())),
                preferred_element_type=jnp.float32,
                precision=lax.Precision.HIGHEST)

    o_ref[...] = coef_ref[1:2, :] + coef_ref[0:1, :] * x_ref[...]


_tc_affine = pl.pallas_call(
    _tc_affine_body,
    grid=(N_SELECTED // COL_BLK, BATCH // ROW_BLK),
    in_specs=[
        pl.BlockSpec((ROW_BLK, COL_BLK), lambda j, i: (i, j)),
        pl.BlockSpec((2, NPAD), lambda j, i: (0, 0)),
        pl.BlockSpec((1, COL_BLK), lambda j, i: (0, j)),
    ],
    out_specs=pl.BlockSpec((ROW_BLK, COL_BLK), lambda j, i: (i, j)),
    out_shape=jax.ShapeDtypeStruct((BATCH, N_SELECTED), jnp.float32),
    scratch_shapes=[pltpu.VMEM((2, COL_BLK), jnp.float32)],
)


def kernel(inputs, input_axon_embeddings, scale, bias, keys_idx):
    keys32 = keys_idx.astype(jnp.int32)
    out_emb = _sc_emb()(input_axon_embeddings, keys32)
    sb = jnp.zeros((2, NPAD), jnp.float32)
    sb = sb.at[0, :N_NEURONS].set(scale).at[1, :N_NEURONS].set(bias)
    out_inputs = _tc_affine(inputs, sb, keys32.reshape(1, N_SELECTED))
    return (out_inputs, out_emb)
